# Initial kernel scaffold; baseline (speedup 1.0000x reference)
import jax
import jax.numpy as jnp
from jax.experimental import pallas as pl


def kernel(edge_index, A_values, H, W, bias):
    raise NotImplementedError("write your pallas kernel here")


# speedup vs baseline: 3.4741x; 3.4741x over previous
import functools

import jax
import jax.numpy as jnp
from jax import lax
from jax.experimental import pallas as pl
from jax.experimental.pallas import tpu as pltpu
from jax.experimental.pallas import tpu_sc as plsc

NC = 2
NS = 16
NW = NC * NS
CHUNK = 128
LANES = 16


def _matmul_tc(h, w):
    n, d_in = h.shape
    d_out = w.shape[1]
    blk = 2000
    grid = n // blk

    def mm_body(h_ref, w_ref, o_ref):
        o_ref[...] = jnp.dot(h_ref[...], w_ref[...],
                             preferred_element_type=jnp.float32)

    return pl.pallas_call(
        mm_body,
        grid=(grid,),
        in_specs=[
            pl.BlockSpec((blk, d_in), lambda i: (i, 0)),
            pl.BlockSpec((d_in, d_out), lambda i: (0, 0)),
        ],
        out_specs=pl.BlockSpec((blk, d_out), lambda i: (i, 0)),
        out_shape=jax.ShapeDtypeStruct((n, d_out), jnp.float32),
    )(h, w)


def _spmm_sc(src, dst, aval, hw, zeros):
    n, d = hw.shape
    e_pad = src.shape[0]
    per_w = e_pad // NW
    n_chunks = per_w // CHUNK
    zr_tiles = 10
    zrows = n // zr_tiles

    mesh = plsc.VectorSubcoreMesh(core_axis_name="c", subcore_axis_name="s",
                                  num_cores=NC, num_subcores=NS)

    @functools.partial(
        pl.kernel,
        out_type=jax.ShapeDtypeStruct((NC, n, d), jnp.float32),
        mesh=mesh,
        compiler_params=pltpu.CompilerParams(needs_layout_passes=False),
        scratch_types=[
            pltpu.VMEM((CHUNK,), jnp.int32),
            pltpu.VMEM((CHUNK,), jnp.int32),
            pltpu.VMEM((CHUNK,), jnp.float32),
            pltpu.VMEM((CHUNK, d), jnp.float32),
            pltpu.VMEM_SHARED((n, d), jnp.float32),
            pltpu.SemaphoreType.DMA,
        ],
    )
    def spmm(src_hbm, dst_hbm, a_hbm, hw_hbm, zero_hbm, out_hbm,
             src_v, dst_v, a_v, rows_v, acc_sh, sem):
        cid = lax.axis_index("c")
        sid = lax.axis_index("s")
        wid = cid * NS + sid

        r0 = sid * zrows

        @pl.when(sid < zr_tiles)
        def _zero():
            pltpu.sync_copy(zero_hbm.at[pl.ds(r0, zrows)],
                            acc_sh.at[pl.ds(r0, zrows)])

        plsc.subcore_barrier()

        base = wid * per_w

        def chunk_body(i, carry):
            off = base + i * CHUNK
            pltpu.sync_copy(src_hbm.at[pl.ds(off, CHUNK)], src_v)
            pltpu.sync_copy(dst_hbm.at[pl.ds(off, CHUNK)], dst_v)
            pltpu.sync_copy(a_hbm.at[pl.ds(off, CHUNK)], a_v)
            pltpu.async_copy(hw_hbm.at[src_v], rows_v, sem).wait()

            def row_body(r, c2):
                a_splat = plsc.load_gather(
                    a_v, [jnp.full((LANES,), 0, jnp.int32) + r])
                for c in range(d // LANES):
                    v = rows_v[r, pl.ds(c * LANES, LANES)]
                    rows_v[r, pl.ds(c * LANES, LANES)] = v * a_splat
                return c2

            lax.fori_loop(0, CHUNK, row_body, 0, unroll=False)
            pltpu.sync_copy(rows_v, acc_sh.at[dst_v], add=True)
            return carry

        lax.fori_loop(0, n_chunks, chunk_body, 0, unroll=False)

        plsc.subcore_barrier()

        @pl.when(sid < zr_tiles)
        def _writeout():
            pltpu.sync_copy(acc_sh.at[pl.ds(r0, zrows)],
                            out_hbm.at[cid, pl.ds(r0, zrows)])

    return spmm(src, dst, aval, hw, zeros)


def _combine_tc(p0, p1, bias2d):
    n, d = p0.shape
    blk = 2000
    grid = n // blk

    def fin_body(a_ref, b_ref, bias_ref, o_ref):
        o_ref[...] = jnp.maximum(a_ref[...] + b_ref[...] + bias_ref[...], 0.0)

    return pl.pallas_call(
        fin_body,
        grid=(grid,),
        in_specs=[
            pl.BlockSpec((blk, d), lambda i: (i, 0)),
            pl.BlockSpec((blk, d), lambda i: (i, 0)),
            pl.BlockSpec((1, d), lambda i: (0, 0)),
        ],
        out_specs=pl.BlockSpec((blk, d), lambda i: (i, 0)),
        out_shape=jax.ShapeDtypeStruct((n, d), jnp.float32),
    )(p0, p1, bias2d)


@jax.jit
def kernel(edge_index, A_values, H, W, bias):
    n = H.shape[0]
    e = A_values.shape[0]
    src = edge_index[1].astype(jnp.int32)
    dst = edge_index[0].astype(jnp.int32)
    aval = A_values.astype(jnp.float32)

    per_w = ((e + NW - 1) // NW + CHUNK - 1) // CHUNK * CHUNK
    e_pad = per_w * NW
    pad = e_pad - e
    if pad:
        src = jnp.concatenate([src, jnp.zeros((pad,), jnp.int32)])
        dst = jnp.concatenate([dst, jnp.zeros((pad,), jnp.int32)])
        aval = jnp.concatenate([aval, jnp.zeros((pad,), jnp.float32)])

    hw = _matmul_tc(H, W)
    zeros = jnp.zeros((n, W.shape[1]), jnp.float32)
    partial = _spmm_sc(src, dst, aval, hw, zeros)
    out = _combine_tc(partial[0], partial[1], bias.reshape(1, -1))
    return out


# speedup vs baseline: 6.5405x; 1.8827x over previous
import functools

import jax
import jax.numpy as jnp
from jax import lax
from jax.experimental import pallas as pl
from jax.experimental.pallas import tpu as pltpu
from jax.experimental.pallas import tpu_sc as plsc

NC = 2
NS = 16
NW = NC * NS
LANES = 16


def _matmul_tc(h, w):
    n, d_in = h.shape
    d_out = w.shape[1]
    blk = 2000
    grid = n // blk

    def mm_body(h_ref, w_ref, o_ref):
        o_ref[...] = jnp.dot(h_ref[...], w_ref[...],
                             preferred_element_type=jnp.float32)

    return pl.pallas_call(
        mm_body,
        grid=(grid,),
        in_specs=[
            pl.BlockSpec((blk, d_in), lambda i: (i, 0)),
            pl.BlockSpec((d_in, d_out), lambda i: (0, 0)),
        ],
        out_specs=pl.BlockSpec((blk, d_out), lambda i: (i, 0)),
        out_shape=jax.ShapeDtypeStruct((n, d_out), jnp.float32),
    )(h, w)


NBUF = 3
CHUNK = 120
GROUP = 3
PAIR = 2 * GROUP * CHUNK


def _spmm_sc(edge9, hw, zeros):
    n, d = hw.shape
    _, n_groups, _, _ = edge9.shape
    n_chunks = n_groups * GROUP
    n_bodies = n_groups // 2
    zr_tiles = 10
    zrows = n // zr_tiles

    mesh = plsc.VectorSubcoreMesh(core_axis_name="c", subcore_axis_name="s",
                                  num_cores=NC, num_subcores=NS)

    @functools.partial(
        pl.kernel,
        out_type=jax.ShapeDtypeStruct((NC, n, d), jnp.float32),
        mesh=mesh,
        compiler_params=pltpu.CompilerParams(needs_layout_passes=False),
        scratch_types=[
            pltpu.VMEM((2 * 9, CHUNK), jnp.int32),
            pltpu.VMEM((NBUF, CHUNK, d), jnp.float32),
            pltpu.VMEM_SHARED((n, d), jnp.float32),
            pltpu.SemaphoreType.DMA((2,)),
            pltpu.SemaphoreType.DMA((NBUF,)),
            pltpu.SemaphoreType.DMA((NBUF,)),
        ],
    )
    def spmm(edge_hbm, hw_hbm, zero_hbm, out_hbm,
             ebuf, rows, acc_sh, esem, gsem, ssem):
        cid = lax.axis_index("c")
        sid = lax.axis_index("s")
        wid = cid * NS + sid

        r0 = sid * zrows

        @pl.when(sid < zr_tiles)
        def _zero():
            pltpu.sync_copy(zero_hbm.at[pl.ds(r0, zrows)],
                            acc_sh.at[pl.ds(r0, zrows)])

        plsc.subcore_barrier()

        def q_of(p, cc, field):
            return p * 9 + cc * 3 + field

        pltpu.sync_copy(edge_hbm.at[wid, 0], ebuf.at[pl.ds(0, 9)])
        for cc in range(GROUP):
            pltpu.async_copy(hw_hbm.at[ebuf.at[q_of(0, cc, 0)]],
                             rows.at[cc], gsem.at[cc])

        def scale_chunk(q_a, b):
            def row_body(r, c2):
                bits = plsc.load_gather(
                    ebuf, [jnp.full((LANES,), q_a, jnp.int32),
                           jnp.full((LANES,), 0, jnp.int32) + r])
                a_splat = plsc.bitcast(bits, jnp.float32)
                for c in range(d // LANES):
                    v = rows[b, r, pl.ds(c * LANES, LANES)]
                    rows[b, r, pl.ds(c * LANES, LANES)] = v * a_splat
                return c2

            lax.fori_loop(0, CHUNK, row_body, 0, unroll=2)

        def drain_scatter(b):
            pltpu.make_async_copy(hw_hbm.at[pl.ds(0, CHUNK)],
                                  rows.at[b], ssem.at[b]).wait()

        def drain_edges(p):
            pltpu.make_async_copy(edge_hbm.at[wid, 0],
                                  ebuf.at[pl.ds(9 * p, 9)], esem.at[p]).wait()

        def body(h, carry):
            for k in range(2 * GROUP):
                i = h * (2 * GROUP) + k
                b = k % NBUF
                nb = (k + 1) % NBUF
                p = k // GROUP
                k1 = (k + 1) % (2 * GROUP)
                p1 = k1 // GROUP
                cc1 = k1 % GROUP

                @pl.when((i >= NBUF - 1) & (i + 1 < n_chunks))
                def _prefetch():
                    drain_scatter(nb)
                    if k in (GROUP - 1, 2 * GROUP - 1):
                        drain_edges(p1)
                    pltpu.async_copy(hw_hbm.at[ebuf.at[q_of(p1, cc1, 0)]],
                                     rows.at[nb], gsem.at[nb])

                if k == 1:
                    pltpu.async_copy(edge_hbm.at[wid, 2 * h + 1],
                                     ebuf.at[pl.ds(9, 9)], esem.at[1])
                if k == GROUP + 1:
                    @pl.when(2 * h + 2 < n_groups)
                    def _eprefetch():
                        pltpu.async_copy(edge_hbm.at[wid, 2 * h + 2],
                                         ebuf.at[pl.ds(0, 9)], esem.at[0])

                pltpu.make_async_copy(hw_hbm.at[pl.ds(0, CHUNK)],
                                      rows.at[b], gsem.at[b]).wait()
                scale_chunk(q_of(p, k % GROUP, 2), b)
                pltpu.async_copy(rows.at[b],
                                 acc_sh.at[ebuf.at[q_of(p, k % GROUP, 1)]],
                                 ssem.at[b], add=True)
            return carry

        lax.fori_loop(0, n_bodies, body, 0, unroll=False)

        for b in range(NBUF):
            drain_scatter(b)

        plsc.subcore_barrier()

        @pl.when(sid < zr_tiles)
        def _writeout():
            pltpu.sync_copy(acc_sh.at[pl.ds(r0, zrows)],
                            out_hbm.at[cid, pl.ds(r0, zrows)])

    return spmm(edge9, hw, zeros)


def _combine_tc(p0, p1, bias2d):
    n, d = p0.shape
    blk = 2000
    grid = n // blk

    def fin_body(a_ref, b_ref, bias_ref, o_ref):
        o_ref[...] = jnp.maximum(a_ref[...] + b_ref[...] + bias_ref[...], 0.0)

    return pl.pallas_call(
        fin_body,
        grid=(grid,),
        in_specs=[
            pl.BlockSpec((blk, d), lambda i: (i, 0)),
            pl.BlockSpec((blk, d), lambda i: (i, 0)),
            pl.BlockSpec((1, d), lambda i: (0, 0)),
        ],
        out_specs=pl.BlockSpec((blk, d), lambda i: (i, 0)),
        out_shape=jax.ShapeDtypeStruct((n, d), jnp.float32),
    )(p0, p1, bias2d)


@jax.jit
def kernel(edge_index, A_values, H, W, bias):
    n = H.shape[0]
    e = A_values.shape[0]
    src = edge_index[1].astype(jnp.int32)
    dst = edge_index[0].astype(jnp.int32)
    aval = A_values.astype(jnp.float32)

    per_w = ((e + NW - 1) // NW + PAIR - 1) // PAIR * PAIR
    e_pad = per_w * NW
    pad = e_pad - e
    if pad:
        src = jnp.concatenate([src, jnp.zeros((pad,), jnp.int32)])
        dst = jnp.concatenate([dst, jnp.zeros((pad,), jnp.int32)])
        aval = jnp.concatenate([aval, jnp.zeros((pad,), jnp.float32)])
    n_groups = per_w // (GROUP * CHUNK)
    a_bits = lax.bitcast_convert_type(aval, jnp.int32)
    fields = jnp.stack([src, dst, a_bits], axis=0)
    fields = fields.reshape(3, NW, n_groups, GROUP, CHUNK)
    edge9 = fields.transpose(1, 2, 3, 0, 4).reshape(NW, n_groups, 9, CHUNK)

    hw = _matmul_tc(H, W)
    zeros = jnp.zeros((n, W.shape[1]), jnp.float32)
    partial = _spmm_sc(edge9, hw, zeros)
    out = _combine_tc(partial[0], partial[1], bias.reshape(1, -1))
    return out


# speedup vs baseline: 6.6886x; 1.0226x over previous
import functools

import jax
import jax.numpy as jnp
from jax import lax
from jax.experimental import pallas as pl
from jax.experimental.pallas import tpu as pltpu
from jax.experimental.pallas import tpu_sc as plsc

NC = 2
NS = 16
NW = NC * NS
LANES = 16


def _matmul_tc(h, w):
    n, d_in = h.shape
    d_out = w.shape[1]
    blk = 2000
    grid = n // blk

    def mm_body(h_ref, w_ref, o_ref):
        o_ref[...] = jnp.dot(h_ref[...], w_ref[...],
                             preferred_element_type=jnp.float32)

    return pl.pallas_call(
        mm_body,
        grid=(grid,),
        in_specs=[
            pl.BlockSpec((blk, d_in), lambda i: (i, 0)),
            pl.BlockSpec((d_in, d_out), lambda i: (0, 0)),
        ],
        out_specs=pl.BlockSpec((blk, d_out), lambda i: (i, 0)),
        out_shape=jax.ShapeDtypeStruct((n, d_out), jnp.float32),
    )(h, w)


NBUF = 3
CHUNK = 120
GROUP = 3
PAIR = 2 * GROUP * CHUNK


def _spmm_sc(edge9, hw, zeros):
    n, d = hw.shape
    _, n_groups, _, _ = edge9.shape
    n_chunks = n_groups * GROUP
    n_bodies = n_groups // 2
    zr_tiles = 10
    zrows = n // zr_tiles

    mesh = plsc.VectorSubcoreMesh(core_axis_name="c", subcore_axis_name="s",
                                  num_cores=NC, num_subcores=NS)

    @functools.partial(
        pl.kernel,
        out_type=jax.ShapeDtypeStruct((NC, n, d), jnp.float32),
        mesh=mesh,
        compiler_params=pltpu.CompilerParams(needs_layout_passes=False),
        scratch_types=[
            pltpu.VMEM((2 * 9, CHUNK), jnp.int32),
            pltpu.VMEM((NBUF, CHUNK, d), jnp.float32),
            pltpu.VMEM_SHARED((n, d), jnp.float32),
            pltpu.SemaphoreType.DMA((2,)),
            pltpu.SemaphoreType.DMA((NBUF,)),
            pltpu.SemaphoreType.DMA((NBUF,)),
        ],
    )
    def spmm(edge_hbm, hw_hbm, zero_hbm, out_hbm,
             ebuf, rows, acc_sh, esem, gsem, ssem):
        cid = lax.axis_index("c")
        sid = lax.axis_index("s")
        wid = cid * NS + sid

        r0 = sid * zrows

        @pl.when(sid < zr_tiles)
        def _zero():
            pltpu.sync_copy(zero_hbm.at[pl.ds(r0, zrows)],
                            acc_sh.at[pl.ds(r0, zrows)])

        plsc.subcore_barrier()

        def q_of(p, cc, field):
            return p * 9 + cc * 3 + field

        pltpu.sync_copy(edge_hbm.at[wid, 0], ebuf.at[pl.ds(0, 9)])
        for cc in range(GROUP):
            pltpu.async_copy(hw_hbm.at[ebuf.at[q_of(0, cc, 0)]],
                             rows.at[cc], gsem.at[cc])

        def scale_chunk(q_a, b):
            @plsc.parallel_loop(0, CHUNK, 1, unroll=4)
            def row_body(r):
                bits = plsc.load_gather(
                    ebuf, [jnp.full((LANES,), q_a, jnp.int32),
                           jnp.full((LANES,), 0, jnp.int32) + r])
                a_splat = plsc.bitcast(bits, jnp.float32)
                for c in range(d // LANES):
                    v = rows[b, r, pl.ds(c * LANES, LANES)]
                    rows[b, r, pl.ds(c * LANES, LANES)] = v * a_splat

        def drain_scatter(b):
            pltpu.make_async_copy(hw_hbm.at[pl.ds(0, CHUNK)],
                                  rows.at[b], ssem.at[b]).wait()

        def drain_edges(p):
            pltpu.make_async_copy(edge_hbm.at[wid, 0],
                                  ebuf.at[pl.ds(9 * p, 9)], esem.at[p]).wait()

        def body(h, carry):
            for k in range(2 * GROUP):
                i = h * (2 * GROUP) + k
                b = k % NBUF
                nb = (k + 1) % NBUF
                p = k // GROUP
                k1 = (k + 1) % (2 * GROUP)
                p1 = k1 // GROUP
                cc1 = k1 % GROUP

                @pl.when((i >= NBUF - 1) & (i + 1 < n_chunks))
                def _prefetch():
                    drain_scatter(nb)
                    if k in (GROUP - 1, 2 * GROUP - 1):
                        drain_edges(p1)
                    pltpu.async_copy(hw_hbm.at[ebuf.at[q_of(p1, cc1, 0)]],
                                     rows.at[nb], gsem.at[nb])

                if k == 1:
                    pltpu.async_copy(edge_hbm.at[wid, 2 * h + 1],
                                     ebuf.at[pl.ds(9, 9)], esem.at[1])
                if k == GROUP + 1:
                    @pl.when(2 * h + 2 < n_groups)
                    def _eprefetch():
                        pltpu.async_copy(edge_hbm.at[wid, 2 * h + 2],
                                         ebuf.at[pl.ds(0, 9)], esem.at[0])

                pltpu.make_async_copy(hw_hbm.at[pl.ds(0, CHUNK)],
                                      rows.at[b], gsem.at[b]).wait()
                scale_chunk(q_of(p, k % GROUP, 2), b)
                pltpu.async_copy(rows.at[b],
                                 acc_sh.at[ebuf.at[q_of(p, k % GROUP, 1)]],
                                 ssem.at[b], add=True)
            return carry

        lax.fori_loop(0, n_bodies, body, 0, unroll=False)

        for b in range(NBUF):
            drain_scatter(b)

        plsc.subcore_barrier()

        @pl.when(sid < zr_tiles)
        def _writeout():
            pltpu.sync_copy(acc_sh.at[pl.ds(r0, zrows)],
                            out_hbm.at[cid, pl.ds(r0, zrows)])

    return spmm(edge9, hw, zeros)


def _combine_tc(p0, p1, bias2d):
    n, d = p0.shape
    blk = 2000
    grid = n // blk

    def fin_body(a_ref, b_ref, bias_ref, o_ref):
        o_ref[...] = jnp.maximum(a_ref[...] + b_ref[...] + bias_ref[...], 0.0)

    return pl.pallas_call(
        fin_body,
        grid=(grid,),
        in_specs=[
            pl.BlockSpec((blk, d), lambda i: (i, 0)),
            pl.BlockSpec((blk, d), lambda i: (i, 0)),
            pl.BlockSpec((1, d), lambda i: (0, 0)),
        ],
        out_specs=pl.BlockSpec((blk, d), lambda i: (i, 0)),
        out_shape=jax.ShapeDtypeStruct((n, d), jnp.float32),
    )(p0, p1, bias2d)


@jax.jit
def kernel(edge_index, A_values, H, W, bias):
    n = H.shape[0]
    e = A_values.shape[0]
    src = edge_index[1].astype(jnp.int32)
    dst = edge_index[0].astype(jnp.int32)
    aval = A_values.astype(jnp.float32)

    per_w = ((e + NW - 1) // NW + PAIR - 1) // PAIR * PAIR
    e_pad = per_w * NW
    pad = e_pad - e
    if pad:
        src = jnp.concatenate([src, jnp.zeros((pad,), jnp.int32)])
        dst = jnp.concatenate([dst, jnp.zeros((pad,), jnp.int32)])
        aval = jnp.concatenate([aval, jnp.zeros((pad,), jnp.float32)])
    n_groups = per_w // (GROUP * CHUNK)
    a_bits = lax.bitcast_convert_type(aval, jnp.int32)
    fields = jnp.stack([src, dst, a_bits], axis=0)
    fields = fields.reshape(3, NW, n_groups, GROUP, CHUNK)
    edge9 = fields.transpose(1, 2, 3, 0, 4).reshape(NW, n_groups, 9, CHUNK)

    hw = _matmul_tc(H, W)
    zeros = jnp.zeros((n, W.shape[1]), jnp.float32)
    partial = _spmm_sc(edge9, hw, zeros)
    out = _combine_tc(partial[0], partial[1], bias.reshape(1, -1))
    return out


# speedup vs baseline: 10.6608x; 1.5939x over previous
import functools

import jax
import jax.numpy as jnp
from jax import lax
from jax.experimental import pallas as pl
from jax.experimental.pallas import tpu as pltpu
from jax.experimental.pallas import tpu_sc as plsc

NC = 2
NS = 16
NW = NC * NS
LANES = 16


def _matmul_tc(h, w):
    n, d_in = h.shape
    d_out = w.shape[1]
    blk = 2000
    grid = n // blk

    def mm_body(h_ref, w_ref, o_ref):
        o_ref[...] = jnp.dot(h_ref[...], w_ref[...],
                             preferred_element_type=jnp.float32)

    return pl.pallas_call(
        mm_body,
        grid=(grid,),
        in_specs=[
            pl.BlockSpec((blk, d_in), lambda i: (i, 0)),
            pl.BlockSpec((d_in, d_out), lambda i: (0, 0)),
        ],
        out_specs=pl.BlockSpec((blk, d_out), lambda i: (i, 0)),
        out_shape=jax.ShapeDtypeStruct((n, d_out), jnp.float32),
    )(h, w)


NBUF = 3
CHUNK = 120
GROUP = 3
PAIR = 2 * GROUP * CHUNK


def _spmm_sc(edge9, hw, zeros):
    n, d = hw.shape
    _, n_groups, _, _ = edge9.shape
    n_chunks = n_groups * GROUP
    n_bodies = n_groups // 2
    zr_tiles = 10
    zrows = n // zr_tiles

    mesh = plsc.VectorSubcoreMesh(core_axis_name="c", subcore_axis_name="s",
                                  num_cores=NC, num_subcores=NS)

    @functools.partial(
        pl.kernel,
        out_type=jax.ShapeDtypeStruct((NC, n, d), jnp.float32),
        mesh=mesh,
        compiler_params=pltpu.CompilerParams(needs_layout_passes=False),
        scratch_types=[
            pltpu.VMEM((2 * 9, CHUNK), jnp.int32),
            pltpu.VMEM((NBUF, CHUNK, d), jnp.float32),
            pltpu.VMEM_SHARED((n, d), jnp.float32),
            pltpu.SemaphoreType.DMA((2,)),
            pltpu.SemaphoreType.DMA((NBUF,)),
            pltpu.SemaphoreType.DMA((NBUF,)),
        ],
    )
    def spmm(edge_hbm, hw_hbm, zero_hbm, out_hbm,
             ebuf, rows, acc_sh, esem, gsem, ssem):
        cid = lax.axis_index("c")
        sid = lax.axis_index("s")
        wid = cid * NS + sid

        r0 = sid * zrows

        @pl.when(sid < zr_tiles)
        def _zero():
            pltpu.sync_copy(zero_hbm.at[pl.ds(r0, zrows)],
                            acc_sh.at[pl.ds(r0, zrows)])

        plsc.subcore_barrier()

        def q_of(p, cc, field):
            return p * 9 + cc * 3 + field

        pltpu.sync_copy(edge_hbm.at[wid, 0], ebuf.at[pl.ds(0, 9)])
        for cc in range(GROUP):
            pltpu.async_copy(hw_hbm.at[ebuf.at[q_of(0, cc, 0)]],
                             rows.at[cc], gsem.at[cc])

        def scale_chunk(q_a, b):
            @plsc.parallel_loop(0, CHUNK, 1, unroll=4)
            def row_body(r):
                bits = plsc.load_gather(
                    ebuf, [jnp.full((LANES,), q_a, jnp.int32),
                           jnp.full((LANES,), 0, jnp.int32) + r])
                a_splat = plsc.bitcast(bits, jnp.float32)
                for c in range(d // LANES):
                    v = rows[b, r, pl.ds(c * LANES, LANES)]
                    rows[b, r, pl.ds(c * LANES, LANES)] = v * a_splat

        def drain_scatter(b):
            pltpu.make_async_copy(hw_hbm.at[pl.ds(0, CHUNK)],
                                  rows.at[b], ssem.at[b]).wait()

        def drain_edges(p):
            pltpu.make_async_copy(edge_hbm.at[wid, 0],
                                  ebuf.at[pl.ds(9 * p, 9)], esem.at[p]).wait()

        def body(h, carry):
            for k in range(2 * GROUP):
                i = h * (2 * GROUP) + k
                b = k % NBUF
                nb = (k + 1) % NBUF
                p = k // GROUP
                k1 = (k + 1) % (2 * GROUP)
                p1 = k1 // GROUP
                cc1 = k1 % GROUP

                @pl.when((i >= NBUF - 1) & (i + 1 < n_chunks))
                def _prefetch():
                    drain_scatter(nb)
                    if k in (GROUP - 1, 2 * GROUP - 1):
                        drain_edges(p1)
                    pltpu.async_copy(hw_hbm.at[ebuf.at[q_of(p1, cc1, 0)]],
                                     rows.at[nb], gsem.at[nb])

                if k == 1:
                    pltpu.async_copy(edge_hbm.at[wid, 2 * h + 1],
                                     ebuf.at[pl.ds(9, 9)], esem.at[1])
                if k == GROUP + 1:
                    @pl.when(2 * h + 2 < n_groups)
                    def _eprefetch():
                        pltpu.async_copy(edge_hbm.at[wid, 2 * h + 2],
                                         ebuf.at[pl.ds(0, 9)], esem.at[0])

                pltpu.make_async_copy(hw_hbm.at[pl.ds(0, CHUNK)],
                                      rows.at[b], gsem.at[b]).wait()
                scale_chunk(q_of(p, k % GROUP, 2), b)
                pltpu.async_copy(rows.at[b],
                                 acc_sh.at[ebuf.at[q_of(p, k % GROUP, 1)]],
                                 ssem.at[b], add=True)
            return carry

        lax.fori_loop(0, n_bodies, body, 0, unroll=False)

        for b in range(NBUF):
            drain_scatter(b)

        plsc.subcore_barrier()

        @pl.when(sid < zr_tiles)
        def _writeout():
            pltpu.sync_copy(acc_sh.at[pl.ds(r0, zrows)],
                            out_hbm.at[cid, pl.ds(r0, zrows)])

    return spmm(edge9, hw, zeros)


def _combine_tc(p0, p1, bias2d):
    n, d = p0.shape
    blk = 2000
    grid = n // blk

    def fin_body(a_ref, b_ref, bias_ref, o_ref):
        o_ref[...] = jnp.maximum(a_ref[...] + b_ref[...] + bias_ref[...], 0.0)

    return pl.pallas_call(
        fin_body,
        grid=(grid,),
        in_specs=[
            pl.BlockSpec((blk, d), lambda i: (i, 0)),
            pl.BlockSpec((blk, d), lambda i: (i, 0)),
            pl.BlockSpec((1, d), lambda i: (0, 0)),
        ],
        out_specs=pl.BlockSpec((blk, d), lambda i: (i, 0)),
        out_shape=jax.ShapeDtypeStruct((n, d), jnp.float32),
    )(p0, p1, bias2d)


@jax.jit
def kernel(edge_index, A_values, H, W, bias):
    n = H.shape[0]
    e = A_values.shape[0]
    src = edge_index[1].astype(jnp.int32)
    dst = edge_index[0].astype(jnp.int32)
    aval = A_values.astype(jnp.float32)

    per_w = ((e + NW - 1) // NW + PAIR - 1) // PAIR * PAIR
    e_pad = per_w * NW
    pad = e_pad - e
    if pad:
        spread = (jnp.arange(pad, dtype=jnp.int32) * 8) % n
        src = jnp.concatenate([src, spread])
        dst = jnp.concatenate([dst, spread])
        aval = jnp.concatenate([aval, jnp.zeros((pad,), jnp.float32)])
    n_groups = per_w // (GROUP * CHUNK)
    a_bits = lax.bitcast_convert_type(aval, jnp.int32)
    fields = jnp.stack([src, dst, a_bits], axis=0)
    fields = fields.reshape(3, NW, n_groups, GROUP, CHUNK)
    edge9 = fields.transpose(1, 2, 3, 0, 4).reshape(NW, n_groups, 9, CHUNK)

    hw = _matmul_tc(H, W)
    zeros = jnp.zeros((n, W.shape[1]), jnp.float32)
    partial = _spmm_sc(edge9, hw, zeros)
    out = _combine_tc(partial[0], partial[1], bias.reshape(1, -1))
    return out


# speedup vs baseline: 11.5382x; 1.0823x over previous
import functools

import jax
import jax.numpy as jnp
from jax import lax
from jax.experimental import pallas as pl
from jax.experimental.pallas import tpu as pltpu
from jax.experimental.pallas import tpu_sc as plsc

NC = 2
NS = 16
NW = NC * NS
LANES = 16

NBUF = 3
CHUNK = 120
GROUP = 3
PAIR = 2 * GROUP * CHUNK


def _matmul_tc(h, w):
    n, d_in = h.shape
    d_out = w.shape[1]
    blk = 2000
    grid = n // blk

    def mm_body(h_ref, w_ref, o_ref):
        o_ref[...] = jnp.dot(h_ref[...], w_ref[...],
                             preferred_element_type=jnp.float32)

    return pl.pallas_call(
        mm_body,
        grid=(grid,),
        in_specs=[
            pl.BlockSpec((blk, d_in), lambda i: (i, 0)),
            pl.BlockSpec((d_in, d_out), lambda i: (0, 0)),
        ],
        out_specs=pl.BlockSpec((blk, d_out), lambda i: (i, 0)),
        out_shape=jax.ShapeDtypeStruct((n, d_out), jnp.float32),
    )(h, w)


def _spmm_sc(src4, dst4, a4, hw, zeros):
    n, d = hw.shape
    _, n_groups, _, _ = src4.shape
    n_chunks = n_groups * GROUP
    n_bodies = n_groups // 2
    zr_tiles = 10
    zrows = n // zr_tiles

    mesh = plsc.VectorSubcoreMesh(core_axis_name="c", subcore_axis_name="s",
                                  num_cores=NC, num_subcores=NS)

    @functools.partial(
        pl.kernel,
        out_type=jax.ShapeDtypeStruct((NC, n, d), jnp.float32),
        mesh=mesh,
        compiler_params=pltpu.CompilerParams(needs_layout_passes=False),
        scratch_types=[
            pltpu.VMEM((2 * GROUP, CHUNK), jnp.int32),
            pltpu.VMEM((2 * GROUP, CHUNK), jnp.int32),
            pltpu.VMEM((2 * GROUP, CHUNK), jnp.float32),
            pltpu.VMEM((NBUF, CHUNK, d), jnp.float32),
            pltpu.VMEM_SHARED((n, d), jnp.float32),
            pltpu.SemaphoreType.DMA((2,)),
            pltpu.SemaphoreType.DMA((NBUF,)),
            pltpu.SemaphoreType.DMA((NBUF,)),
        ],
    )
    def spmm(src_hbm, dst_hbm, a_hbm, hw_hbm, zero_hbm, out_hbm,
             sbuf, dbuf, abuf, rows, acc_sh, esem, gsem, ssem):
        cid = lax.axis_index("c")
        sid = lax.axis_index("s")
        wid = cid * NS + sid

        r0 = sid * zrows

        @pl.when(sid < zr_tiles)
        def _zero():
            pltpu.sync_copy(zero_hbm.at[pl.ds(r0, zrows)],
                            acc_sh.at[pl.ds(r0, zrows)])

        plsc.subcore_barrier()

        def edge_fetch(g, p, sem):
            sl = pl.ds(p * GROUP, GROUP)
            pltpu.async_copy(src_hbm.at[wid, g], sbuf.at[sl], sem)
            pltpu.async_copy(dst_hbm.at[wid, g], dbuf.at[sl], sem)
            pltpu.async_copy(a_hbm.at[wid, g], abuf.at[sl], sem)

        def drain_edges(p):
            sl = pl.ds(p * GROUP, GROUP)
            for buf in (sbuf, dbuf, abuf):
                pltpu.make_async_copy(src_hbm.at[wid, 0],
                                      buf.at[sl], esem.at[p]).wait()

        edge_fetch(0, 0, esem.at[0])
        drain_edges(0)
        for cc in range(GROUP):
            pltpu.async_copy(hw_hbm.at[sbuf.at[cc]], rows.at[cc],
                             gsem.at[cc])

        def scale_chunk(p, cc, b):
            q = p * GROUP + cc

            @plsc.parallel_loop(0, CHUNK, 1, unroll=4)
            def row_body(r):
                a_splat = plsc.load_gather(
                    abuf, [jnp.full((LANES,), q, jnp.int32),
                           jnp.full((LANES,), 0, jnp.int32) + r])
                for c in range(d // LANES):
                    v = rows[b, r, pl.ds(c * LANES, LANES)]
                    rows[b, r, pl.ds(c * LANES, LANES)] = v * a_splat

        def drain_scatter(b):
            pltpu.make_async_copy(hw_hbm.at[pl.ds(0, CHUNK)],
                                  rows.at[b], ssem.at[b]).wait()

        def body(h, carry):
            for k in range(2 * GROUP):
                i = h * (2 * GROUP) + k
                b = k % NBUF
                nb = (k + 1) % NBUF
                p = k // GROUP
                k1 = (k + 1) % (2 * GROUP)
                p1 = k1 // GROUP
                cc1 = k1 % GROUP

                @pl.when((i >= NBUF - 1) & (i + 1 < n_chunks))
                def _prefetch():
                    drain_scatter(nb)
                    if k in (GROUP - 1, 2 * GROUP - 1):
                        drain_edges(p1)
                    pltpu.async_copy(hw_hbm.at[sbuf.at[p1 * GROUP + cc1]],
                                     rows.at[nb], gsem.at[nb])

                if k == 1:
                    edge_fetch(2 * h + 1, 1, esem.at[1])
                if k == GROUP + 1:
                    @pl.when(2 * h + 2 < n_groups)
                    def _eprefetch():
                        edge_fetch(2 * h + 2, 0, esem.at[0])

                pltpu.make_async_copy(hw_hbm.at[pl.ds(0, CHUNK)],
                                      rows.at[b], gsem.at[b]).wait()
                scale_chunk(p, k % GROUP, b)
                pltpu.async_copy(rows.at[b],
                                 acc_sh.at[dbuf.at[p * GROUP + k % GROUP]],
                                 ssem.at[b], add=True)
            return carry

        lax.fori_loop(0, n_bodies, body, 0, unroll=False)

        for b in range(NBUF):
            drain_scatter(b)

        plsc.subcore_barrier()

        @pl.when(sid < zr_tiles)
        def _writeout():
            pltpu.sync_copy(acc_sh.at[pl.ds(r0, zrows)],
                            out_hbm.at[cid, pl.ds(r0, zrows)])

    return spmm(src4, dst4, a4, hw, zeros)


def _combine_tc(partial, bias2d):
    _, n, d = partial.shape
    blk = 2000
    grid = n // blk

    def fin_body(p_ref, bias_ref, o_ref):
        o_ref[...] = jnp.maximum(
            p_ref[0] + p_ref[1] + bias_ref[...], 0.0)

    return pl.pallas_call(
        fin_body,
        grid=(grid,),
        in_specs=[
            pl.BlockSpec((2, blk, d), lambda i: (0, i, 0)),
            pl.BlockSpec((1, d), lambda i: (0, 0)),
        ],
        out_specs=pl.BlockSpec((blk, d), lambda i: (i, 0)),
        out_shape=jax.ShapeDtypeStruct((n, d), jnp.float32),
    )(partial, bias2d)


@jax.jit
def kernel(edge_index, A_values, H, W, bias):
    n = H.shape[0]
    e = A_values.shape[0]
    src = edge_index[1].astype(jnp.int32)
    dst = edge_index[0].astype(jnp.int32)
    aval = A_values.astype(jnp.float32)

    per_w = ((e + NW - 1) // NW + PAIR - 1) // PAIR * PAIR
    e_pad = per_w * NW
    pad = e_pad - e
    if pad:
        spread = (jnp.arange(pad, dtype=jnp.int32) * 8) % n
        src = jnp.concatenate([src, spread])
        dst = jnp.concatenate([dst, spread])
        aval = jnp.concatenate([aval, jnp.zeros((pad,), jnp.float32)])
    n_groups = per_w // (GROUP * CHUNK)
    src4 = src.reshape(NW, n_groups, GROUP, CHUNK)
    dst4 = dst.reshape(NW, n_groups, GROUP, CHUNK)
    a4 = aval.reshape(NW, n_groups, GROUP, CHUNK)

    hw = _matmul_tc(H, W)
    zeros = jnp.zeros((n, W.shape[1]), jnp.float32)
    partial = _spmm_sc(src4, dst4, a4, hw, zeros)
    out = _combine_tc(partial, bias.reshape(1, -1))
    return out
